# Initial kernel scaffold; baseline (speedup 1.0000x reference)
import jax
import jax.numpy as jnp
from jax.experimental import pallas as pl


def kernel(X, edge_index, edge_weight, W, b):
    raise NotImplementedError("write your pallas kernel here")


# speedup vs baseline: 6.6184x; 6.6184x over previous
import functools

import jax
import jax.numpy as jnp
from jax import lax
from jax.experimental import pallas as pl
from jax.experimental.pallas import tpu as pltpu
from jax.experimental.pallas import tpu_sc as plsc

N_NODES = 10000
D = 128
N_EDGES = 320000
NC = 2
NS = 16
NW = NC * NS
K = 80
B = 25
NB = N_EDGES // (NW * B * K)
N_PAD = 10240
SROWS = N_PAD // NS

_mesh = plsc.VectorSubcoreMesh(core_axis_name="c", subcore_axis_name="s")


@functools.partial(
    pl.kernel,
    out_type=jax.ShapeDtypeStruct((NC, N_PAD, D), jnp.float32),
    mesh=_mesh,
    scratch_types=[
        pltpu.VMEM((B, K), jnp.int32),
        pltpu.VMEM((B, K), jnp.int32),
        pltpu.VMEM((B, K), jnp.float32),
        pltpu.VMEM((K, D), jnp.float32),
        pltpu.VMEM_SHARED((N_PAD, D), jnp.float32),
        pltpu.SemaphoreType.DMA,
    ],
)
def _sc_aggregate(x_hbm, src_hbm, dst_hbm, ew_hbm, part_hbm,
                  srcv, dstv, ewv, rows, acc, sem):
    c = lax.axis_index("c")
    s = lax.axis_index("s")
    wid = c * NS + s

    def _zrow(r, carry):
        for q in range(D // 16):
            rows[r, pl.ds(q * 16, 16)] = jnp.zeros((16,), jnp.float32)
        return carry
    lax.fori_loop(0, K, _zrow, 0)
    for i in range(SROWS // K):
        pltpu.sync_copy(rows, acc.at[pl.ds(s * SROWS + i * K, K)])
    plsc.subcore_barrier()

    def _block(bb, carry):
        pltpu.sync_copy(src_hbm.at[wid, bb], srcv)
        pltpu.sync_copy(dst_hbm.at[wid, bb], dstv)
        pltpu.sync_copy(ew_hbm.at[wid, bb], ewv)

        def _chunk(j, carry2):
            pltpu.async_copy(x_hbm.at[srcv.at[j]], rows, sem).wait()

            def _grp(g, carry3):
                wv = ewv[j, pl.ds(g * 16, 16)]
                for l in range(16):
                    w = jnp.full((16,), wv[l], dtype=jnp.float32)
                    e = g * 16 + l
                    for q in range(D // 16):
                        sl = pl.ds(q * 16, 16)
                        rows[e, sl] = rows[e, sl] * w
                return carry3
            lax.fori_loop(0, K // 16, _grp, 0)

            pltpu.sync_copy(rows, acc.at[dstv.at[j]], add=True)
            return carry2
        lax.fori_loop(0, B, _chunk, 0)
        return carry
    lax.fori_loop(0, NB, _block, 0)
    plsc.subcore_barrier()

    for i in range(SROWS // K):
        r0 = s * SROWS + i * K
        pltpu.sync_copy(acc.at[pl.ds(r0, K)], rows)
        pltpu.sync_copy(rows, part_hbm.at[c, pl.ds(r0, K)])


_RB = 1000


def _tc_body(p_ref, wt_ref, b_ref, o_ref):
    x = p_ref[0] + p_ref[1]
    o_ref[...] = (
        jnp.dot(x, wt_ref[...], preferred_element_type=jnp.float32) + b_ref[...]
    )


def _tc_linear(parts, wt, b2):
    return pl.pallas_call(
        _tc_body,
        out_shape=jax.ShapeDtypeStruct((N_NODES, D), jnp.float32),
        grid=(N_NODES // _RB,),
        in_specs=[
            pl.BlockSpec((NC, _RB, D), lambda i: (0, i, 0)),
            pl.BlockSpec((D, D), lambda i: (0, 0)),
            pl.BlockSpec((1, D), lambda i: (0, 0)),
        ],
        out_specs=pl.BlockSpec((_RB, D), lambda i: (i, 0)),
    )(parts, wt, b2)


@jax.jit
def _run(X, src4d, dst4d, ew4d, wt, b2):
    parts = _sc_aggregate(X, src4d, dst4d, ew4d)
    return _tc_linear(parts, wt, b2)


def kernel(X, edge_index, edge_weight, W, b):
    src4d = edge_index[0].astype(jnp.int32).reshape(NW, NB, B, K)
    dst4d = edge_index[1].astype(jnp.int32).reshape(NW, NB, B, K)
    ew4d = edge_weight.reshape(NW, NB, B, K)
    return _run(X, src4d, dst4d, ew4d, W.T, b.reshape(1, D))


# speedup vs baseline: 8.5341x; 1.2894x over previous
import functools

import jax
import jax.numpy as jnp
from jax import lax
from jax.experimental import pallas as pl
from jax.experimental.pallas import tpu as pltpu
from jax.experimental.pallas import tpu_sc as plsc

N_NODES = 10000
D = 128
N_EDGES = 320000
NC = 2
NS = 16
NW = NC * NS
K = 64
BLK = 32
NBLK = 5
CH = BLK * NBLK
E_PAD = NW * CH * K
N_PAD = 10240
SROWS = N_PAD // NS

_mesh = plsc.VectorSubcoreMesh(core_axis_name="c", subcore_axis_name="s")


@functools.partial(
    pl.kernel,
    out_type=jax.ShapeDtypeStruct((NC, N_PAD, D), jnp.float32),
    mesh=_mesh,
    scratch_types=[
        pltpu.VMEM((2, BLK, K), jnp.int32),
        pltpu.VMEM((BLK, K), jnp.float32),
        pltpu.VMEM((K, D), jnp.float32),
        pltpu.VMEM((K, D), jnp.float32),
        pltpu.VMEM_SHARED((N_PAD, D), jnp.float32),
        pltpu.SemaphoreType.DMA,
        pltpu.SemaphoreType.DMA,
    ],
)
def _sc_aggregate(x_hbm, ed_hbm, ew_hbm, part_hbm,
                  ib, wb, rows_a, rows_b, acc, sem_a, sem_b):
    c = lax.axis_index("c")
    s = lax.axis_index("s")
    wid = c * NS + s

    def _zrow(r, carry):
        for q in range(D // 16):
            rows_a[r, pl.ds(q * 16, 16)] = jnp.zeros((16,), jnp.float32)
        return carry
    lax.fori_loop(0, K, _zrow, 0)
    for i in range(SROWS // K):
        pltpu.sync_copy(rows_a, acc.at[pl.ds(s * SROWS + i * K, K)])
    plsc.subcore_barrier()

    def _scale(rows, l):
        def _grp(g, carry):
            wv = wb[l, pl.ds(g * 16, 16)]
            for lane in range(16):
                w = jnp.full((16,), wv[lane], dtype=jnp.float32)
                e = g * 16 + lane
                for q in range(D // 16):
                    sl = pl.ds(q * 16, 16)
                    rows[e, sl] = rows[e, sl] * w
            return carry
        lax.fori_loop(0, K // 16, _grp, 0)

    def _block(bb, carry):
        pltpu.sync_copy(ed_hbm.at[wid, bb], ib)
        pltpu.sync_copy(ew_hbm.at[wid, bb], wb)
        pltpu.async_copy(x_hbm.at[ib.at[0, 0]], rows_a, sem_a)

        def _pair(t, carry2):
            j0 = 2 * t
            j1 = 2 * t + 1
            pltpu.make_async_copy(
                x_hbm.at[ib.at[0, j0]], rows_a, sem_a).wait()
            pltpu.async_copy(x_hbm.at[ib.at[0, j1]], rows_b, sem_b)
            _scale(rows_a, j0)
            pltpu.sync_copy(rows_a, acc.at[ib.at[1, j0]], add=True)
            pltpu.make_async_copy(
                x_hbm.at[ib.at[0, j1]], rows_b, sem_b).wait()

            @pl.when(t < BLK // 2 - 1)
            def _():
                pltpu.async_copy(x_hbm.at[ib.at[0, j1 + 1]], rows_a, sem_a)
            _scale(rows_b, j1)
            pltpu.sync_copy(rows_b, acc.at[ib.at[1, j1]], add=True)
            return carry2
        lax.fori_loop(0, BLK // 2, _pair, 0)
        return carry
    lax.fori_loop(0, NBLK, _block, 0)
    plsc.subcore_barrier()

    for i in range(SROWS // K):
        r0 = s * SROWS + i * K
        pltpu.sync_copy(acc.at[pl.ds(r0, K)], rows_a)
        pltpu.sync_copy(rows_a, part_hbm.at[c, pl.ds(r0, K)])


_RB = 1000


def _tc_body(p_ref, wt_ref, b_ref, o_ref):
    x = p_ref[0] + p_ref[1]
    o_ref[...] = (
        jnp.dot(x, wt_ref[...], preferred_element_type=jnp.float32) + b_ref[...]
    )


def _tc_linear(parts, wt, b2):
    return pl.pallas_call(
        _tc_body,
        out_shape=jax.ShapeDtypeStruct((N_NODES, D), jnp.float32),
        grid=(N_NODES // _RB,),
        in_specs=[
            pl.BlockSpec((NC, _RB, D), lambda i: (0, i, 0)),
            pl.BlockSpec((D, D), lambda i: (0, 0)),
            pl.BlockSpec((1, D), lambda i: (0, 0)),
        ],
        out_specs=pl.BlockSpec((_RB, D), lambda i: (i, 0)),
    )(parts, wt, b2)


@jax.jit
def _run(X, ed, ew4, wt, b2):
    parts = _sc_aggregate(X, ed, ew4)
    return _tc_linear(parts, wt, b2)


def kernel(X, edge_index, edge_weight, W, b):
    pad = E_PAD - N_EDGES
    pad_ids = jnp.arange(pad, dtype=jnp.int32)
    src = jnp.concatenate(
        [edge_index[0].astype(jnp.int32), pad_ids % N_NODES])
    dst = jnp.concatenate(
        [edge_index[1].astype(jnp.int32),
         N_NODES + pad_ids % (N_PAD - N_NODES)])
    ew4 = jnp.concatenate(
        [edge_weight, jnp.zeros((pad,), jnp.float32)]).reshape(NW, NBLK, BLK, K)
    ed = jnp.stack([src.reshape(NW, NBLK, BLK, K),
                    dst.reshape(NW, NBLK, BLK, K)], axis=2)
    return _run(X, ed, ew4, W.T, b.reshape(1, D))
